# baseline (speedup 1.0000x reference)
import jax
import jax.numpy as jnp
from jax.experimental import pallas as pl
from jax.experimental.pallas import tpu as pltpu


def _dma_body(xu_ref, xi_ref, ou_ref, oi_ref, sem_u, sem_i):
    cu = pltpu.make_async_copy(xu_ref, ou_ref, sem_u)
    ci = pltpu.make_async_copy(xi_ref, oi_ref, sem_i)
    cu.start()
    ci.start()
    cu.wait()
    ci.wait()


def kernel(x_user, x_item, edge_index):
    del edge_index
    n, d = x_user.shape
    ou, oi = pl.pallas_call(
        _dma_body,
        in_specs=[
            pl.BlockSpec(memory_space=pl.ANY),
            pl.BlockSpec(memory_space=pl.ANY),
        ],
        out_specs=[
            pl.BlockSpec(memory_space=pl.ANY),
            pl.BlockSpec(memory_space=pl.ANY),
        ],
        out_shape=[
            jax.ShapeDtypeStruct((n, d), x_user.dtype),
            jax.ShapeDtypeStruct((n, d), x_item.dtype),
        ],
        scratch_shapes=[pltpu.SemaphoreType.DMA, pltpu.SemaphoreType.DMA],
    )(x_user, x_item)
    return (ou, oi)


# speedup vs baseline: 26.4232x; 26.4232x over previous
import jax
import jax.numpy as jnp
from jax.experimental import pallas as pl
from jax.experimental.pallas import tpu as pltpu


def _copy_body(xu_ref, xi_ref, ou_ref, oi_ref):
    ou_ref[...] = xu_ref[...]
    oi_ref[...] = xi_ref[...]


def kernel(x_user, x_item, edge_index):
    del edge_index
    n, d = x_user.shape
    blk = 1000
    grid = (n // blk,)
    spec = pl.BlockSpec((blk, d), lambda i: (i, 0))
    ou, oi = pl.pallas_call(
        _copy_body,
        grid=grid,
        in_specs=[spec, spec],
        out_specs=[spec, spec],
        out_shape=[
            jax.ShapeDtypeStruct((n, d), x_user.dtype),
            jax.ShapeDtypeStruct((n, d), x_item.dtype),
        ],
    )(x_user, x_item)
    return (ou, oi)


# speedup vs baseline: 40.7621x; 1.5427x over previous
import jax
import jax.numpy as jnp
from jax.experimental import pallas as pl
from jax.experimental.pallas import tpu as pltpu


def _copy_body(xu_ref, xi_ref, ou_ref, oi_ref):
    ou_ref[...] = xu_ref[...]
    oi_ref[...] = xi_ref[...]


def kernel(x_user, x_item, edge_index):
    del edge_index
    n, d = x_user.shape
    blk = 5000
    grid = (n // blk,)
    spec = pl.BlockSpec((blk, d), lambda i: (i, 0))
    ou, oi = pl.pallas_call(
        _copy_body,
        grid=grid,
        in_specs=[spec, spec],
        out_specs=[spec, spec],
        out_shape=[
            jax.ShapeDtypeStruct((n, d), x_user.dtype),
            jax.ShapeDtypeStruct((n, d), x_item.dtype),
        ],
    )(x_user, x_item)
    return (ou, oi)
